# baseline (speedup 1.0000x reference)
import functools

import jax
import jax.numpy as jnp
from jax import lax
from jax.experimental import pallas as pl
from jax.experimental.pallas import tpu as pltpu
from jax.experimental.pallas import tpu_sc as plsc

_F = 26
_V = 100001
_VS = 100000
_D = 32
_B = 16384
_R = _B * _F
_NC = 2
_NS = 16
_NW = _NC * _NS
_RPW = _R // _NW
_CR = 128
_G = _RPW // _CR


def _stage_body(t_ref, o_ref, sem):
    copies = [
        pltpu.make_async_copy(
            t_ref.at[t, pl.ds(0, _VS), :],
            o_ref.at[pl.ds(t * _VS, _VS), :],
            sem,
        )
        for t in range(_F)
    ]
    for c in copies:
        c.start()
    for c in copies:
        c.wait()


_stage = pl.pallas_call(
    _stage_body,
    out_shape=jax.ShapeDtypeStruct((_F * _VS, _D), jnp.float32),
    in_specs=[pl.BlockSpec(memory_space=pltpu.MemorySpace.HBM)],
    out_specs=pl.BlockSpec(memory_space=pltpu.MemorySpace.HBM),
    scratch_shapes=[pltpu.SemaphoreType.DMA],
)


@functools.partial(
    pl.kernel,
    out_type=jax.ShapeDtypeStruct((_R, _D), jnp.float32),
    mesh=plsc.VectorSubcoreMesh(core_axis_name="c", subcore_axis_name="s"),
    scratch_types=[
        pltpu.VMEM((_RPW,), jnp.int32),
        pltpu.VMEM((_G, _CR), jnp.int32),
        pltpu.VMEM((_CR, _D), jnp.float32),
        pltpu.SemaphoreType.DMA,
    ],
    compiler_params=pltpu.CompilerParams(use_tc_tiling_on_sc=False),
)
def _emb_gather(x_hbm, tab_hbm, out_hbm, idx_a, idx_v, rows_v, sem):
    wid = lax.axis_index("s") * _NC + lax.axis_index("c")
    base = wid * _RPW
    pltpu.sync_copy(x_hbm.at[pl.ds(base, _RPW)], idx_a)

    def body(g, carry):
        for k in range(_CR // 16):
            j = g * _CR + k * 16
            p = lax.iota(jnp.int32, 16) + j
            f = lax.rem(p, _F)
            idx_v[g, pl.ds(k * 16, 16)] = idx_a[pl.ds(j, 16)] + f * _VS
        pltpu.async_copy(tab_hbm.at[idx_v.at[g]], rows_v, sem).wait()
        pltpu.sync_copy(rows_v, out_hbm.at[pl.ds(base + g * _CR, _CR)])
        return carry

    lax.fori_loop(0, _G, body, 0)


def kernel(x, tables):
    x1 = x.reshape(_R)
    tab2 = _stage(tables)
    out = _emb_gather(x1, tab2)
    return out.reshape(_B, _F * _D)


# speedup vs baseline: 1.6876x; 1.6876x over previous
import functools

import jax
import jax.numpy as jnp
from jax import lax
from jax.experimental import pallas as pl
from jax.experimental.pallas import tpu as pltpu
from jax.experimental.pallas import tpu_sc as plsc

_F = 26
_V = 100001
_VS = 100000
_D = 32
_B = 16384
_R = _B * _F
_NC = 2
_NS = 16
_NW = _NC * _NS
_RPW = _R // _NW
_CR = 128
_G = _RPW // _CR


@functools.partial(
    pl.kernel,
    out_type=jax.ShapeDtypeStruct((_R, _D), jnp.float32),
    mesh=plsc.VectorSubcoreMesh(core_axis_name="c", subcore_axis_name="s"),
    scratch_types=[
        pltpu.VMEM((_RPW,), jnp.int32),
        pltpu.VMEM((_G, _CR), jnp.int32),
        pltpu.VMEM((_CR, _D), jnp.float32),
        pltpu.SemaphoreType.DMA,
    ],
    compiler_params=pltpu.CompilerParams(use_tc_tiling_on_sc=False),
)
def _emb_gather(x_hbm, tab_hbm, out_hbm, idx_a, idx_v, rows_v, sem):
    wid = lax.axis_index("s") * _NC + lax.axis_index("c")
    base = wid * _RPW
    pltpu.sync_copy(x_hbm.at[pl.ds(base, _RPW)], idx_a)

    def body(g, carry):
        for k in range(_CR // 16):
            j = g * _CR + k * 16
            p = lax.iota(jnp.int32, 16) + j
            f = lax.rem(p, _F)
            idx_v[g, pl.ds(k * 16, 16)] = idx_a[pl.ds(j, 16)] + f * _VS
        pltpu.async_copy(tab_hbm.at[idx_v.at[g]], rows_v, sem).wait()
        pltpu.sync_copy(rows_v, out_hbm.at[pl.ds(base + g * _CR, _CR)])
        return carry

    lax.fori_loop(0, _G, body, 0)


def kernel(x, tables):
    x1 = x.reshape(_R)
    tab2 = jnp.take(tables, jnp.arange(_VS), axis=1).reshape(_F * _VS, _D)
    out = _emb_gather(x1, tab2)
    return out.reshape(_B, _F * _D)


# speedup vs baseline: 30.0643x; 17.8147x over previous
import functools

import jax
import jax.numpy as jnp
from jax import lax
from jax.experimental import pallas as pl
from jax.experimental.pallas import tpu as pltpu
from jax.experimental.pallas import tpu_sc as plsc

_F = 26
_V = 100001
_VS = 100000
_D = 32
_B = 16384
_R = _B * _F
_NC = 2
_NS = 16
_NW = _NC * _NS
_RPW = _R // _NW
_CR = 128
_G = _RPW // _CR


@functools.partial(
    pl.kernel,
    out_type=jax.ShapeDtypeStruct((_R, _D), jnp.float32),
    mesh=plsc.VectorSubcoreMesh(core_axis_name="c", subcore_axis_name="s"),
    scratch_types=[
        pltpu.VMEM((_RPW,), jnp.int32),
        pltpu.VMEM((_G, _CR), jnp.int32),
        pltpu.VMEM((_CR, _D), jnp.float32),
        pltpu.SemaphoreType.DMA,
    ],
    compiler_params=pltpu.CompilerParams(use_tc_tiling_on_sc=False),
)
def _emb_gather(x_hbm, tab_hbm, out_hbm, idx_a, idx_v, rows_v, sem):
    wid = lax.axis_index("s") * _NC + lax.axis_index("c")
    base = wid * _RPW
    pltpu.sync_copy(x_hbm.at[pl.ds(base, _RPW)], idx_a)

    def body(g, carry):
        for k in range(_CR // 16):
            j = g * _CR + k * 16
            p = lax.iota(jnp.int32, 16) + j
            f = lax.rem(p, _F)
            idx_v[g, pl.ds(k * 16, 16)] = idx_a[pl.ds(j, 16)] + f * _VS
        pltpu.async_copy(tab_hbm.at[0].at[idx_v.at[g]], rows_v, sem).wait()
        pltpu.sync_copy(rows_v, out_hbm.at[pl.ds(base + g * _CR, _CR)])
        return carry

    lax.fori_loop(0, _G, body, 0)


def kernel(x, tables):
    x1 = x.reshape(_R)
    tab3 = tables[:, :_VS, :]
    out = _emb_gather(x1, tab3)
    return out.reshape(_B, _F * _D)
